# baseline (speedup 1.0000x reference)
import jax
import jax.numpy as jnp
from jax import lax
from jax.experimental import pallas as pl
from jax.experimental.pallas import tpu as pltpu
from jax.experimental.pallas import tpu_sc as plsc

BATCH = 16384
NOBJ = 50
DIM = 16
NC = 2
NS = 16
NW = NC * NS
ROWS_PER_W = BATCH // NW
CHUNK = 64
NCHUNK = ROWS_PER_W // CHUNK
NOBJ_PAD = 56
IDX_PER_CHUNK = CHUNK * NOBJ_PAD
OUT_PER_CHUNK = CHUNK * (NOBJ - 1)


def _sc_body(idx_hbm, lt_hbm, out_hbm, idx_v, rows_v, out_v, gsem):
    wid = lax.axis_index("s") * NC + lax.axis_index("c")
    iota = lax.iota(jnp.int32, 16)

    for c in range(NCHUNK):
        row_base = wid * ROWS_PER_W + c * CHUNK
        pltpu.sync_copy(idx_hbm.at[pl.ds(row_base, CHUNK), :], idx_v)
        copies = [
            pltpu.async_copy(
                lt_hbm.at[idx_v.at[i, pl.ds(0, NOBJ_PAD)]],
                rows_v.at[pl.ds(i * NOBJ_PAD, NOBJ_PAD)],
                gsem,
            )
            for i in range(CHUNK)
        ]
        for cp in copies:
            cp.wait()

        for g in range(4):
            row0 = (g * 16 + iota) * NOBJ_PAD
            outb = (g * 16 + iota) * (NOBJ - 1)
            svec = [
                plsc.load_gather(rows_v, [row0, (iota + k) & 15])
                for k in range(DIM)
            ]

            @pl.loop(0, NOBJ - 1)
            def _(j, row0=row0, outb=outb, svec=svec):
                orow = row0 + (j + 1)
                acc = None
                for k in range(DIM):
                    o = plsc.load_gather(rows_v, [orow, (iota + k) & 15])
                    t = svec[k] - o
                    acc = t * t if acc is None else acc + t * t
                plsc.store_scatter(out_v, [outb + j], acc)

        pltpu.sync_copy(
            out_v,
            out_hbm.at[pl.ds(wid * ROWS_PER_W * (NOBJ - 1) + c * OUT_PER_CHUNK,
                             OUT_PER_CHUNK)],
        )


@jax.jit
def _run(idx, lt):
    mesh = plsc.VectorSubcoreMesh(core_axis_name="c", subcore_axis_name="s")
    flat = pl.kernel(
        _sc_body,
        out_type=jax.ShapeDtypeStruct((BATCH * (NOBJ - 1),), jnp.float32),
        mesh=mesh,
        scratch_types=[
            pltpu.VMEM((CHUNK, 128), jnp.int32),
            pltpu.VMEM((IDX_PER_CHUNK, DIM), jnp.float32),
            pltpu.VMEM((OUT_PER_CHUNK,), jnp.float32),
            pltpu.SemaphoreType.DMA,
        ],
        compiler_params=pltpu.CompilerParams(
            needs_layout_passes=False,
            use_tc_tiling_on_sc=False,
        ),
    )(idx, lt)
    return flat.reshape(BATCH, NOBJ - 1)


def kernel(inputs, lt):
    idx = jnp.pad(inputs.astype(jnp.int32), ((0, 0), (0, 128 - NOBJ)))
    return _run(idx, lt)


# speedup vs baseline: 1.7718x; 1.7718x over previous
import jax
import jax.numpy as jnp
from jax import lax
from jax.experimental import pallas as pl
from jax.experimental.pallas import tpu as pltpu
from jax.experimental.pallas import tpu_sc as plsc

BATCH = 16384
NOBJ = 50
DIM = 16
NC = 2
NS = 16
NW = NC * NS
ROWS_PER_W = BATCH // NW
CHUNK = 64
NCHUNK = ROWS_PER_W // CHUNK
IDX_PER_CHUNK = CHUNK * NOBJ
OUT_PER_CHUNK = CHUNK * (NOBJ - 1)


def _sc_body(idx_hbm, lt_hbm, out_hbm, idx_v, rows_v, out_v, gsem):
    wid = lax.axis_index("s") * NC + lax.axis_index("c")
    iota = lax.iota(jnp.int32, 16)

    for c in range(NCHUNK):
        row_base = wid * ROWS_PER_W + c * CHUNK
        pltpu.sync_copy(idx_hbm.at[pl.ds(row_base, CHUNK), :], idx_v)
        copies = [
            pltpu.async_copy(
                lt_hbm.at[idx_v.at[i]],
                rows_v.at[pl.ds(i * NOBJ, NOBJ)],
                gsem,
            )
            for i in range(CHUNK)
        ]
        for cp in copies:
            cp.wait()

        for g in range(4):
            row0 = (g * 16 + iota) * NOBJ
            outb = (g * 16 + iota) * (NOBJ - 1)
            svec = [
                plsc.load_gather(rows_v, [row0, (iota + k) & 15])
                for k in range(DIM)
            ]

            @pl.loop(0, NOBJ - 1)
            def _(j, row0=row0, outb=outb, svec=svec):
                orow = row0 + (j + 1)
                acc = None
                for k in range(DIM):
                    o = plsc.load_gather(rows_v, [orow, (iota + k) & 15])
                    t = svec[k] - o
                    acc = t * t if acc is None else acc + t * t
                plsc.store_scatter(out_v, [outb + j], acc)

        pltpu.sync_copy(
            out_v,
            out_hbm.at[pl.ds(wid * ROWS_PER_W * (NOBJ - 1) + c * OUT_PER_CHUNK,
                             OUT_PER_CHUNK)],
        )


@jax.jit
def _run(idx, lt_flat):
    lt = lt_flat.reshape(1000000, DIM)
    mesh = plsc.VectorSubcoreMesh(core_axis_name="c", subcore_axis_name="s")
    flat = pl.kernel(
        _sc_body,
        out_type=jax.ShapeDtypeStruct((BATCH * (NOBJ - 1),), jnp.float32),
        mesh=mesh,
        scratch_types=[
            pltpu.VMEM((CHUNK, NOBJ), jnp.int32),
            pltpu.VMEM((IDX_PER_CHUNK, DIM), jnp.float32),
            pltpu.VMEM((OUT_PER_CHUNK,), jnp.float32),
            pltpu.SemaphoreType.DMA,
        ],
        compiler_params=pltpu.CompilerParams(
            needs_layout_passes=False,
            use_tc_tiling_on_sc=False,
        ),
    )(idx, lt)
    return flat.reshape(BATCH, NOBJ - 1)


def kernel(inputs, lt):
    lt_flat = lax.optimization_barrier(lt.reshape(-1))
    return _run(inputs.astype(jnp.int32), lt_flat)
